# baseline (speedup 1.0000x reference)
import functools

import jax
import jax.numpy as jnp
from jax import lax
from jax.experimental import pallas as pl
from jax.experimental.pallas import tpu as pltpu
from jax.experimental.pallas import tpu_sc as plsc

N_NODES = 10000
D = 128
DW = D // 2
N_EDGES = 320000
NC = 2
NS = 16
NW = NC * NS
EDGES_PER_W = N_EDGES // NW
CHUNK = 80
NCHUNK = EDGES_PER_W // CHUNK
NBUF = 4
ACC_ROWS = 10240
ROWS_PER_SUB = ACC_ROWS // NS
ZROWS = 32
LANES = 16
G16 = CHUNK // LANES


def _mm_body(x_ref, w_ref, o_ref):
    o_ref[...] = jnp.dot(x_ref[...], w_ref[...],
                         preferred_element_type=jnp.float32
                         ).astype(jnp.bfloat16)


def _matmul(x, W):
    M, K = x.shape
    _, N = W.shape
    BM = 2000
    return pl.pallas_call(
        _mm_body,
        grid=(M // BM,),
        in_specs=[pl.BlockSpec((BM, K), lambda i: (i, 0)),
                  pl.BlockSpec((K, N), lambda i: (0, 0))],
        out_specs=pl.BlockSpec((BM, N), lambda i: (i, 0)),
        out_shape=jax.ShapeDtypeStruct((M, N), jnp.bfloat16),
    )(x, W)


def _add_body(p_ref, o_ref):
    o_ref[...] = (p_ref[0].astype(jnp.float32) +
                  p_ref[1].astype(jnp.float32))


def _sum_partials(p):
    M, N = N_NODES, D
    BM = 2000
    return pl.pallas_call(
        _add_body,
        grid=(M // BM,),
        in_specs=[pl.BlockSpec((2, BM, N), lambda i: (0, i, 0))],
        out_specs=pl.BlockSpec((BM, N), lambda i: (i, 0)),
        out_shape=jax.ShapeDtypeStruct((M, N), jnp.float32),
    )(p)


def _sc_body(support_hbm, src_hbm, dst_hbm, w_hbm, out_hbm,
             src_v, dst_v, w_v, rows_v, prod_v, zbuf, acc, semg, sems):
    c = lax.axis_index("c")
    s = lax.axis_index("s")
    wid = c * NS + s
    base = wid * EDGES_PER_W

    pltpu.sync_copy(src_hbm.at[pl.ds(base, EDGES_PER_W)], src_v)

    def _zrow(r, _):
        for j in range(D // 32):
            zbuf[r, pl.ds(j * 32, 32)] = jnp.zeros((32,), jnp.bfloat16)
        return 0
    lax.fori_loop(0, ZROWS, _zrow, 0)
    for z in range(ROWS_PER_SUB // ZROWS):
        pltpu.async_copy(zbuf,
                         acc.at[pl.ds(s * ROWS_PER_SUB + z * ZROWS, ZROWS)],
                         sems.at[NBUF - 1])

    def _gather(i, b):
        pltpu.async_copy(dst_hbm.at[pl.ds(base + i * CHUNK, CHUNK)],
                         dst_v.at[b], semg.at[b])
        pltpu.async_copy(w_hbm.at[pl.ds(base + i * CHUNK, CHUNK)],
                         w_v.at[b], semg.at[b])
        for k in range(G16):
            idx = src_v[pl.ds(i * CHUNK + k * LANES, LANES)]
            pltpu.async_copy(support_hbm.at[idx],
                             rows_v.at[b, pl.ds(k * LANES, LANES)],
                             semg.at[b])

    def _wait_gather(b):
        pltpu.make_async_copy(dst_hbm.at[pl.ds(0, CHUNK)], dst_v.at[b],
                              semg.at[b]).wait()
        pltpu.make_async_copy(w_hbm.at[pl.ds(0, CHUNK)], w_v.at[b],
                              semg.at[b]).wait()
        pltpu.make_async_copy(support_hbm.at[pl.ds(0, CHUNK)], rows_v.at[b],
                              semg.at[b]).wait()

    def _scale(b):
        rb = rows_v.at[b]
        pb = prod_v.at[b]
        wv = w_v.at[b]

        def _edges(u, _):
            for t in range(4):
                e = u * 4 + t
                wb = plsc.load_gather(wv, [jnp.full((LANES,), e, jnp.int32)])
                wb2 = plsc.pack(wb, wb, format=plsc.PackFormat.INTERLEAVED)
                for j in range(D // 32):
                    sl = pl.ds(j * 32, 32)
                    pb[e, sl] = rb[e, sl] * wb2
            return 0
        lax.fori_loop(0, CHUNK // 4, _edges, 0)

    def _scatter(b):
        for k in range(G16):
            idx = dst_v[b, pl.ds(k * LANES, LANES)]
            pltpu.async_copy(prod_v.at[b, pl.ds(k * LANES, LANES)],
                             acc.at[idx], sems.at[b], add=True)

    def _wait_scatter(b):
        pltpu.make_async_copy(prod_v.at[b], acc.at[pl.ds(0, CHUNK)],
                              sems.at[b]).wait()

    _gather(0, 0)
    _gather(1, 1)
    for z in range(ROWS_PER_SUB // ZROWS):
        pltpu.make_async_copy(zbuf, acc.at[pl.ds(0, ZROWS)],
                              sems.at[NBUF - 1]).wait()
    plsc.subcore_barrier()

    _wait_gather(0)
    _scale(0)
    _scatter(0)
    _gather(2, 2)

    _wait_gather(1)
    _scale(1)
    _scatter(1)
    _gather(3, 3)

    def _step(i, b):
        _wait_gather(b)
        _scale(b)
        _scatter(b)
        _wait_scatter((b + 2) % NBUF)

        @pl.when(i + 2 < NCHUNK)
        def _():
            _gather(i + 2, (b + 2) % NBUF)

    def _body(p, _):
        for t in range(NBUF):
            i = p * NBUF + 2 + t
            _step(i, (2 + t) % NBUF)
        return 0

    nfull = (NCHUNK - 2) // NBUF
    lax.fori_loop(0, nfull, _body, 0)

    for i in range(2 + nfull * NBUF, NCHUNK):
        _step(i, i % NBUF)
    _wait_scatter((NCHUNK - 2) % NBUF)
    _wait_scatter((NCHUNK - 1) % NBUF)

    plsc.subcore_barrier()

    pltpu.sync_copy(acc.at[pl.ds(s * ROWS_PER_SUB, ROWS_PER_SUB)],
                    out_hbm.at[c, pl.ds(s * ROWS_PER_SUB, ROWS_PER_SUB)])


def _sc_spmm(support, src, dst, w):
    mesh = plsc.VectorSubcoreMesh(core_axis_name="c", subcore_axis_name="s")
    kfn = functools.partial(
        pl.kernel,
        out_type=jax.ShapeDtypeStruct((NC, ACC_ROWS, D), jnp.bfloat16),
        mesh=mesh,
        compiler_params=pltpu.CompilerParams(needs_layout_passes=False,
                                             use_tc_tiling_on_sc=False),
        scratch_types=[
            pltpu.VMEM((EDGES_PER_W,), jnp.int32),
            pltpu.VMEM((NBUF, CHUNK), jnp.int32),
            pltpu.VMEM((NBUF, CHUNK), jnp.float32),
            pltpu.VMEM((NBUF, CHUNK, D), jnp.bfloat16),
            pltpu.VMEM((NBUF, CHUNK, D), jnp.bfloat16),
            pltpu.VMEM((ZROWS, D), jnp.bfloat16),
            pltpu.VMEM_SHARED((ACC_ROWS, D), jnp.bfloat16),
            pltpu.SemaphoreType.DMA((NBUF,)),
            pltpu.SemaphoreType.DMA((NBUF,)),
        ],
    )(_sc_body)
    return kfn(support, src, dst, w)


def kernel(x, edge_index, edge_weight, W):
    support = _matmul(x, W)
    dst = edge_index[0].astype(jnp.int32)
    src = edge_index[1].astype(jnp.int32)
    partials = _sc_spmm(support, src, dst, edge_weight)
    return _sum_partials(partials)


# speedup vs baseline: 1.6369x; 1.6369x over previous
import functools

import jax
import jax.numpy as jnp
from jax import lax
from jax.experimental import pallas as pl
from jax.experimental.pallas import tpu as pltpu
from jax.experimental.pallas import tpu_sc as plsc

N_NODES = 10000
D = 128
N_EDGES = 320000
NC = 2
NS = 16
NW = NC * NS
EDGES_PER_W = N_EDGES // NW
CHUNK = 80
NCHUNK = EDGES_PER_W // CHUNK
NBUF = 3
ACC_ROWS = 10240
ROWS_PER_SUB = ACC_ROWS // NS
ZROWS = 8
LANES = 16
G16 = CHUNK // LANES


def _mm_sum_body(p_ref, w_ref, o_ref):
    o_ref[...] = jnp.dot(p_ref[0] + p_ref[1], w_ref[...],
                         preferred_element_type=jnp.float32)


def _sum_matmul(p, W):
    M, N = N_NODES, D
    BM = 2000
    return pl.pallas_call(
        _mm_sum_body,
        grid=(M // BM,),
        in_specs=[pl.BlockSpec((2, BM, N), lambda i: (0, i, 0)),
                  pl.BlockSpec((N, N), lambda i: (0, 0))],
        out_specs=pl.BlockSpec((BM, N), lambda i: (i, 0)),
        out_shape=jax.ShapeDtypeStruct((M, N), jnp.float32),
    )(p, W)


def _sc_body(x_hbm, src_hbm, dst_hbm, w_hbm, out_hbm,
             src_v, dst_v, w_v, rows_v, zbuf, acc, semg, sems):
    c = lax.axis_index("c")
    s = lax.axis_index("s")
    wid = c * NS + s
    base = wid * EDGES_PER_W

    pltpu.sync_copy(src_hbm.at[pl.ds(base, EDGES_PER_W)], src_v)

    def _zrow(r, _):
        for j in range(D // LANES):
            zbuf[r, pl.ds(j * LANES, LANES)] = jnp.zeros((LANES,), jnp.float32)
        return 0
    lax.fori_loop(0, ZROWS, _zrow, 0)
    for z in range(ROWS_PER_SUB // ZROWS):
        pltpu.async_copy(zbuf,
                         acc.at[pl.ds(s * ROWS_PER_SUB + z * ZROWS, ZROWS)],
                         sems.at[NBUF - 1])

    def _gather(i, b):
        pltpu.async_copy(dst_hbm.at[pl.ds(base + i * CHUNK, CHUNK)],
                         dst_v.at[b], semg.at[b])
        pltpu.async_copy(w_hbm.at[pl.ds(base + i * CHUNK, CHUNK)],
                         w_v.at[b], semg.at[b])
        for k in range(G16):
            idx = src_v[pl.ds(i * CHUNK + k * LANES, LANES)]
            pltpu.async_copy(x_hbm.at[idx],
                             rows_v.at[b, pl.ds(k * LANES, LANES)],
                             semg.at[b])

    def _wait_gather(b):
        pltpu.make_async_copy(dst_hbm.at[pl.ds(0, CHUNK)], dst_v.at[b],
                              semg.at[b]).wait()
        pltpu.make_async_copy(w_hbm.at[pl.ds(0, CHUNK)], w_v.at[b],
                              semg.at[b]).wait()
        pltpu.make_async_copy(x_hbm.at[pl.ds(0, CHUNK)], rows_v.at[b],
                              semg.at[b]).wait()

    def _scale(b):
        rb = rows_v.at[b]
        wv = w_v.at[b]

        def _edges(u, _):
            for t in range(4):
                e = u * 4 + t
                wb = plsc.load_gather(wv, [jnp.full((LANES,), e, jnp.int32)])
                for j in range(D // LANES):
                    sl = pl.ds(j * LANES, LANES)
                    rb[e, sl] = rb[e, sl] * wb
            return 0
        lax.fori_loop(0, CHUNK // 4, _edges, 0)

    def _scatter(b):
        for k in range(G16):
            idx = dst_v[b, pl.ds(k * LANES, LANES)]
            pltpu.async_copy(rows_v.at[b, pl.ds(k * LANES, LANES)],
                             acc.at[idx], sems.at[b], add=True)

    def _wait_scatter(b):
        pltpu.make_async_copy(rows_v.at[b], acc.at[pl.ds(0, CHUNK)],
                              sems.at[b]).wait()

    _gather(0, 0)
    _gather(1, 1)
    for z in range(ROWS_PER_SUB // ZROWS):
        pltpu.make_async_copy(zbuf, acc.at[pl.ds(0, ZROWS)],
                              sems.at[NBUF - 1]).wait()
    plsc.subcore_barrier()

    _wait_gather(0)
    _scale(0)
    _scatter(0)
    _gather(2, 2)

    def _step(i, b):
        _wait_gather(b)
        _scale(b)
        _scatter(b)
        _wait_scatter((b - 1) % NBUF)

        @pl.when(i + 2 < NCHUNK)
        def _():
            _gather(i + 2, (b + 2) % NBUF)

    def _body(p, _):
        for t in range(NBUF):
            i = p * NBUF + 1 + t
            _step(i, (1 + t) % NBUF)
        return 0

    nfull = (NCHUNK - 1) // NBUF
    lax.fori_loop(0, nfull, _body, 0)

    for i in range(1 + nfull * NBUF, NCHUNK):
        _step(i, i % NBUF)
    _wait_scatter((NCHUNK - 1) % NBUF)

    plsc.subcore_barrier()

    pltpu.sync_copy(acc.at[pl.ds(s * ROWS_PER_SUB, ROWS_PER_SUB)],
                    out_hbm.at[c, pl.ds(s * ROWS_PER_SUB, ROWS_PER_SUB)])


def _sc_spmm(x, src, dst, w):
    mesh = plsc.VectorSubcoreMesh(core_axis_name="c", subcore_axis_name="s")
    kfn = functools.partial(
        pl.kernel,
        out_type=jax.ShapeDtypeStruct((NC, ACC_ROWS, D), jnp.float32),
        mesh=mesh,
        compiler_params=pltpu.CompilerParams(needs_layout_passes=False),
        scratch_types=[
            pltpu.VMEM((EDGES_PER_W,), jnp.int32),
            pltpu.VMEM((NBUF, CHUNK), jnp.int32),
            pltpu.VMEM((NBUF, CHUNK), jnp.float32),
            pltpu.VMEM((NBUF, CHUNK, D), jnp.float32),
            pltpu.VMEM((ZROWS, D), jnp.float32),
            pltpu.VMEM_SHARED((ACC_ROWS, D), jnp.float32),
            pltpu.SemaphoreType.DMA((NBUF,)),
            pltpu.SemaphoreType.DMA((NBUF,)),
        ],
    )(_sc_body)
    return kfn(x, src, dst, w)


def kernel(x, edge_index, edge_weight, W):
    dst = edge_index[0].astype(jnp.int32)
    src = edge_index[1].astype(jnp.int32)
    partials = _sc_spmm(x, src, dst, edge_weight)
    return _sum_matmul(partials, W)
